# Initial kernel scaffold; baseline (speedup 1.0000x reference)
import jax
import jax.numpy as jnp
from jax.experimental import pallas as pl


def kernel(x, edge_index, edge_weight, W_s, W_e):
    raise NotImplementedError("write your pallas kernel here")


# speedup vs baseline: 3.4103x; 3.4103x over previous
import functools

import jax
import jax.numpy as jnp
from jax import lax
from jax.experimental import pallas as pl
from jax.experimental.pallas import tpu as pltpu
from jax.experimental.pallas import tpu_sc as plsc

N = 10000
E = 160000
D = 256
H = D // 2
NC = 2
NS = 16
L = 16
CH = 128
T = -(-E // (NS * CH))
E_PAD = T * NS * CH
ROW_STRIDE = 632
ROW_BASE = 520
ROW_EXTRA = ROW_STRIDE - ROW_BASE


def _sc_agg_body(xcat_hbm, src_hbm, dst_hbm, w_hbm, out_hbm,
                 gidx_v, didx_v, w_v, rows_v, acc_sh, sem):
    c = lax.axis_index("c")
    s = lax.axis_index("s")

    zeros16 = jnp.zeros((L,), jnp.float32)

    def zero_row(i, carry):
        for g in range(H // L):
            rows_v[i, pl.ds(g * L, L)] = zeros16
        return carry

    lax.fori_loop(0, CH, zero_row, 0)

    base = pl.multiple_of(s * ROW_STRIDE, 8)

    def copy_acc_region(dst_is_hbm):
        for j in range(ROW_BASE // CH):
            sl = pl.ds(base + j * CH, CH)
            if dst_is_hbm:
                pltpu.sync_copy(acc_sh.at[sl], out_hbm.at[c, sl])
            else:
                pltpu.sync_copy(rows_v, acc_sh.at[sl])
        sl8 = pl.ds(base + ROW_BASE - 8, 8)
        sle = pl.ds(base + ROW_BASE, ROW_EXTRA)
        if dst_is_hbm:
            pltpu.sync_copy(acc_sh.at[sl8], out_hbm.at[c, sl8])

            @pl.when(s < NS - 1)
            def _():
                pltpu.sync_copy(acc_sh.at[sle], out_hbm.at[c, sle])
        else:
            pltpu.sync_copy(rows_v.at[pl.ds(0, 8)], acc_sh.at[sl8])

            @pl.when(s < NS - 1)
            def _():
                pltpu.sync_copy(rows_v.at[pl.ds(0, ROW_EXTRA)],
                                acc_sh.at[sle])

    copy_acc_region(dst_is_hbm=False)
    plsc.subcore_barrier()

    tbl_off = c * N

    def chunk(t, carry):
        e0 = (s * T + t) * CH
        pltpu.sync_copy(src_hbm.at[pl.ds(e0, CH)], gidx_v)
        pltpu.sync_copy(dst_hbm.at[pl.ds(e0, CH)], didx_v)
        pltpu.sync_copy(w_hbm.at[pl.ds(e0, CH)], w_v)
        for g in range(CH // L):
            sl = pl.ds(g * L, L)
            gidx_v[sl] = gidx_v[sl] + tbl_off
        pltpu.async_copy(xcat_hbm.at[gidx_v], rows_v, sem).wait()

        def scale(b, carry2):
            w16 = w_v[pl.ds(b * L, L)]
            for e in range(L):
                wspl = jnp.full((L,), w16[e], jnp.float32)
                row = b * L + e
                for g in range(H // L):
                    sl = pl.ds(g * L, L)
                    rows_v[row, sl] = rows_v[row, sl] * wspl
            return carry2

        lax.fori_loop(0, CH // L, scale, 0)

        pltpu.sync_copy(rows_v, acc_sh.at[didx_v], add=True)
        return carry

    lax.fori_loop(0, T, chunk, 0)
    plsc.subcore_barrier()

    copy_acc_region(dst_is_hbm=True)


_sc_agg = functools.partial(
    pl.kernel,
    out_type=jax.ShapeDtypeStruct((NC, N, H), jnp.float32),
    mesh=plsc.VectorSubcoreMesh(core_axis_name="c", subcore_axis_name="s"),
    scratch_types=[
        pltpu.VMEM((CH,), jnp.int32),
        pltpu.VMEM((CH,), jnp.int32),
        pltpu.VMEM((CH,), jnp.float32),
        pltpu.VMEM((CH, H), jnp.float32),
        pltpu.VMEM_SHARED((N, H), jnp.float32),
        pltpu.SemaphoreType.DMA,
    ],
)(_sc_agg_body)


def _tc_body(x_ref, al_ref, ar_ref, wst_ref, wsb_ref, we_ref, o_ref):
    acc = jnp.dot(al_ref[...], wst_ref[...], preferred_element_type=jnp.float32)
    acc = acc + jnp.dot(ar_ref[...], wsb_ref[...],
                        preferred_element_type=jnp.float32)
    acc = acc + jnp.dot(x_ref[...], we_ref[...],
                        preferred_element_type=jnp.float32)
    o_ref[...] = jnp.maximum(acc, 0.0)


_MB = 2000


def _tc_fused(x, aggl, aggr, wst, wsb, we):
    return pl.pallas_call(
        _tc_body,
        out_shape=jax.ShapeDtypeStruct((N, D), jnp.float32),
        grid=(N // _MB,),
        in_specs=[
            pl.BlockSpec((_MB, D), lambda i: (i, 0)),
            pl.BlockSpec((_MB, H), lambda i: (i, 0)),
            pl.BlockSpec((_MB, H), lambda i: (i, 0)),
            pl.BlockSpec((H, D), lambda i: (0, 0)),
            pl.BlockSpec((H, D), lambda i: (0, 0)),
            pl.BlockSpec((D, D), lambda i: (0, 0)),
        ],
        out_specs=pl.BlockSpec((_MB, D), lambda i: (i, 0)),
    )(x, aggl, aggr, wst, wsb, we)


def kernel(x, edge_index, edge_weight, W_s, W_e):
    src = edge_index[0].astype(jnp.int32)
    dst = edge_index[1].astype(jnp.int32)
    w = edge_weight.astype(jnp.float32)

    pad = E_PAD - E
    fill = (jnp.arange(pad, dtype=jnp.int32) % N)
    src_p = jnp.concatenate([src, fill])
    dst_p = jnp.concatenate([dst, fill])
    w_p = jnp.concatenate([w, jnp.zeros((pad,), jnp.float32)])

    xcat = jnp.concatenate([x[:, :H], x[:, H:]], axis=0)

    agg = _sc_agg(xcat, src_p, dst_p, w_p)

    return _tc_fused(x, agg[0], agg[1], W_s[:H, :], W_s[H:, :], W_e)
